# Initial kernel scaffold; baseline (speedup 1.0000x reference)
import jax
import jax.numpy as jnp
from jax.experimental import pallas as pl


def kernel(pois_embs, hg_up_rows, hg_up_cols, hg_up_vals, hg_pu_rows, hg_pu_cols, hg_pu_vals):
    raise NotImplementedError("write your pallas kernel here")


# speedup vs baseline: 3.2330x; 3.2330x over previous
import functools

import jax
import jax.numpy as jnp
from jax import lax
from jax.experimental import pallas as pl
from jax.experimental.pallas import tpu as pltpu
from jax.experimental.pallas import tpu_sc as plsc

N_POIS = 10000
N_USERS = 10000
EMB = 128
NNZ = 320000

NC = 2
NS = 16
NW = NC * NS
RPW = 320
NROWS_PAD = NW * RPW
K = 128
EPAD = NNZ + K
SEARCH_ROUNDS = 7
LANES = EMB // 16


def _lower_bound_step(rows, probe, sem, lo, hi, target):
    step = jnp.maximum(1, (hi - lo + 15) // 16)
    iota = lax.iota(jnp.int32, 16)
    q = jnp.minimum(lo + iota * step, jnp.maximum(hi - 1, 0))
    cp = pltpu.make_async_copy(rows.at[q], probe, sem)
    cp.start()
    cp.wait()
    vals = probe[...]
    c = jnp.int32(0)
    for lane in range(16):
        c = c + jnp.where(vals[lane] < target, 1, 0).astype(jnp.int32)
    new_lo = jnp.where(c > 0, jnp.minimum(lo + (c - 1) * step + 1, hi), lo)
    new_hi = jnp.where(c < 16, jnp.minimum(lo + c * step, hi), hi)
    return new_lo, new_hi


def _spmm_body(dense, rows, cols, vals, out,
               probe_a, probe_b, colsb, rowsb, valsb, gath, acc,
               sem_a, sem_b, sem_v, sem_g):
    c = lax.axis_index("c")
    s = lax.axis_index("s")
    wid = s * NC + c
    r0 = wid * RPW
    r1 = r0 + RPW

    def bs_body(_, st):
        lo0, hi0, lo1, hi1 = st
        lo0, hi0 = _lower_bound_step(rows, probe_a, sem_a, lo0, hi0, r0)
        lo1, hi1 = _lower_bound_step(rows, probe_b, sem_b, lo1, hi1, r1)
        return lo0, hi0, lo1, hi1

    z = jnp.int32(0)
    n = jnp.int32(NNZ)
    e0, _, e1, _ = lax.fori_loop(0, SEARCH_ROUNDS, bs_body, (z, n, z, n))

    zero = jnp.zeros((16,), jnp.float32)

    def zero_body(r, carry):
        for cc in range(LANES):
            acc[r, pl.ds(cc * 16, 16)] = zero
        return carry

    lax.fori_loop(0, RPW, zero_body, 0)

    e0a = pl.multiple_of((e0 // 8) * 8, 8)
    nchunks = (e1 - e0a + (K - 1)) // K
    iota16 = lax.iota(jnp.int32, 16)

    def chunk_body(ci, carry):
        base = pl.multiple_of(e0a + ci * K, 8)
        cpc = pltpu.make_async_copy(cols.at[pl.ds(base, K)], colsb, sem_a)
        cpr = pltpu.make_async_copy(rows.at[pl.ds(base, K)], rowsb, sem_b)
        cpv = pltpu.make_async_copy(vals.at[pl.ds(base, K)], valsb, sem_v)
        cpc.start()
        cpr.start()
        cpv.start()
        cpc.wait()
        cpr.wait()
        cpv.wait()
        cpg = pltpu.make_async_copy(dense.at[colsb], gath, sem_g)
        cpg.start()
        cpg.wait()

        def group_body(g16, gcarry):
            j0 = g16 * 16
            vv = valsb[pl.ds(j0, 16)]
            rr = rowsb[pl.ds(j0, 16)]
            eidx = base + j0 + iota16
            vvz = jnp.where((eidx >= e0) & (eidx < e1), vv, 0.0)
            rcl = jnp.clip(rr - r0, 0, RPW - 1)
            for lane in range(16):
                v_s = vvz[lane]
                r_s = rcl[lane]
                jrow = j0 + lane
                for cc in range(LANES):
                    gv = gath[jrow, pl.ds(cc * 16, 16)]
                    plsc.addupdate(acc.at[r_s, pl.ds(cc * 16, 16)], gv * v_s)
            return gcarry

        lax.fori_loop(0, K // 16, group_body, 0)
        return carry

    lax.fori_loop(0, nchunks, chunk_body, 0)

    pltpu.sync_copy(acc, out.at[pl.ds(r0, RPW)])


@functools.cache
def _spmm_kernel(n_dense_rows):
    mesh = plsc.VectorSubcoreMesh(core_axis_name="c", subcore_axis_name="s")
    return pl.kernel(
        _spmm_body,
        mesh=mesh,
        out_type=jax.ShapeDtypeStruct((NROWS_PAD, EMB), jnp.float32),
        scratch_types=[
            pltpu.VMEM((16,), jnp.int32),
            pltpu.VMEM((16,), jnp.int32),
            pltpu.VMEM((K,), jnp.int32),
            pltpu.VMEM((K,), jnp.int32),
            pltpu.VMEM((K,), jnp.float32),
            pltpu.VMEM((K, EMB), jnp.float32),
            pltpu.VMEM((RPW, EMB), jnp.float32),
            pltpu.SemaphoreType.DMA,
            pltpu.SemaphoreType.DMA,
            pltpu.SemaphoreType.DMA,
            pltpu.SemaphoreType.DMA,
        ],
    )


def _spmm(dense, rows, cols, vals):
    pad = EPAD - NNZ
    rows_p = jnp.concatenate(
        [rows.astype(jnp.int32), jnp.full((pad,), NROWS_PAD, jnp.int32)])
    cols_p = jnp.concatenate([cols.astype(jnp.int32), jnp.zeros((pad,), jnp.int32)])
    vals_p = jnp.concatenate([vals, jnp.zeros((pad,), jnp.float32)])
    return _spmm_kernel(dense.shape[0])(dense, rows_p, cols_p, vals_p)


def kernel(pois_embs, hg_up_rows, hg_up_cols, hg_up_vals,
           hg_pu_rows, hg_pu_cols, hg_pu_vals):
    msg = _spmm(pois_embs, hg_up_rows, hg_up_cols, hg_up_vals)
    out = _spmm(msg, hg_pu_rows, hg_pu_cols, hg_pu_vals)
    return out[:N_POIS]


# speedup vs baseline: 3.9411x; 1.2190x over previous
import functools

import jax
import jax.numpy as jnp
from jax import lax
from jax.experimental import pallas as pl
from jax.experimental.pallas import tpu as pltpu
from jax.experimental.pallas import tpu_sc as plsc

N_POIS = 10000
N_USERS = 10000
EMB = 128
NNZ = 320000

NC = 2
NS = 16
NW = NC * NS
RPW = 320
NROWS_PAD = NW * RPW
K = 128
EPAD = NNZ + 640
SEARCH_ROUNDS = 7
LANES = EMB // 16


def _lower_bound_step(rows, probe, sem, lo, hi, target):
    step = jnp.maximum(1, (hi - lo + 15) // 16)
    q = jnp.minimum(lo + lax.iota(jnp.int32, 16) * step,
                    jnp.maximum(hi - 1, 0))
    cp = pltpu.make_async_copy(rows.at[q], probe, sem)
    cp.start()
    cp.wait()
    vals = probe[...]
    c = jnp.int32(0)
    for lane in range(16):
        c = c + jnp.where(vals[lane] < target, 1, 0).astype(jnp.int32)
    new_lo = jnp.where(c > 0, jnp.minimum(lo + (c - 1) * step + 1, hi), lo)
    new_hi = jnp.where(c < 16, jnp.minimum(lo + c * step, hi), hi)
    return new_lo, new_hi


def _spmm_body(dense, rows, cols, vals, out,
               probe_a, probe_b,
               cbuf0, cbuf1, rbuf0, rbuf1, vbuf0, vbuf1, gath0, gath1, acc,
               sem_a, sem_b,
               semc0, semc1, semr0, semr1, semv0, semv1, semg0, semg1):
    c = lax.axis_index("c")
    s = lax.axis_index("s")
    wid = s * NC + c
    r0 = wid * RPW
    r1 = r0 + RPW

    cbuf = (cbuf0, cbuf1)
    rbuf = (rbuf0, rbuf1)
    vbuf = (vbuf0, vbuf1)
    gath = (gath0, gath1)
    semc = (semc0, semc1)
    semr = (semr0, semr1)
    semv = (semv0, semv1)
    semg = (semg0, semg1)

    zero = jnp.zeros((16,), jnp.float32)

    def zero_body(r, carry):
        for cc in range(LANES):
            acc[r, pl.ds(cc * 16, 16)] = zero
        return carry

    lax.fori_loop(0, RPW, zero_body, 0)

    def bs_body(_, st):
        lo0, hi0, lo1, hi1 = st
        lo0, hi0 = _lower_bound_step(rows, probe_a, sem_a, lo0, hi0, r0)
        lo1, hi1 = _lower_bound_step(rows, probe_b, sem_b, lo1, hi1, r1)
        return lo0, hi0, lo1, hi1

    z = jnp.int32(0)
    n = jnp.int32(NNZ)
    e0, _, e1, _ = lax.fori_loop(0, SEARCH_ROUNDS, bs_body, (z, n, z, n))

    e0a = pl.multiple_of((e0 // 8) * 8, 8)
    nchunks = (e1 - e0a + (K - 1)) // K
    npairs = jnp.maximum(1, (nchunks + 1) // 2)
    iota16 = lax.iota(jnp.int32, 16)

    def cbase(ci):
        return pl.multiple_of(e0a + ci * K, 8)

    def start_cols(ci, b):
        pltpu.make_async_copy(cols.at[pl.ds(cbase(ci), K)], cbuf[b], semc[b]).start()

    def start_rv(ci, b):
        pltpu.make_async_copy(rows.at[pl.ds(cbase(ci), K)], rbuf[b], semr[b]).start()
        pltpu.make_async_copy(vals.at[pl.ds(cbase(ci), K)], vbuf[b], semv[b]).start()

    def start_gather(b):
        pltpu.make_async_copy(dense.at[cbuf[b]], gath[b], semg[b]).start()

    def wait_cols(b):
        pltpu.make_async_copy(cols.at[pl.ds(0, K)], cbuf[b], semc[b]).wait()

    def wait_rv(b):
        pltpu.make_async_copy(rows.at[pl.ds(0, K)], rbuf[b], semr[b]).wait()
        pltpu.make_async_copy(vals.at[pl.ds(0, K)], vbuf[b], semv[b]).wait()

    def wait_gather(b):
        pltpu.make_async_copy(dense.at[cbuf[b]], gath[b], semg[b]).wait()

    def accumulate(ci, b):
        base = cbase(ci)

        def group_body(g16, gcarry):
            j0 = g16 * 16
            vv = vbuf[b][pl.ds(j0, 16)]
            rr = rbuf[b][pl.ds(j0, 16)]
            eidx = base + j0 + iota16
            vvz = jnp.where((eidx >= e0) & (eidx < e1), vv, 0.0)
            rcl = jnp.clip(rr - r0, 0, RPW - 1)
            for lane in range(16):
                v_s = vvz[lane]
                r_s = rcl[lane]
                jrow = j0 + lane
                for cc in range(LANES):
                    gv = gath[b][jrow, pl.ds(cc * 16, 16)]
                    plsc.addupdate(acc.at[r_s, pl.ds(cc * 16, 16)], gv * v_s)
            return gcarry

        lax.fori_loop(0, K // 16, group_body, 0)

    start_cols(0, 0)
    wait_cols(0)
    start_gather(0)
    start_rv(0, 0)
    start_cols(1, 1)
    start_rv(1, 1)

    def pair_body(p, carry):
        ci = p * 2
        for b in (0, 1):
            wait_cols(1 - b)
            start_gather(1 - b)
            wait_gather(b)
            wait_rv(b)
            accumulate(ci + b, b)
            start_cols(ci + b + 2, b)
            start_rv(ci + b + 2, b)
        return carry

    lax.fori_loop(0, npairs, pair_body, 0)

    wait_gather(0)
    wait_cols(1)
    wait_rv(0)
    wait_rv(1)

    pltpu.sync_copy(acc, out.at[pl.ds(r0, RPW)])


@functools.cache
def _spmm_kernel(n_dense_rows):
    mesh = plsc.VectorSubcoreMesh(core_axis_name="c", subcore_axis_name="s")
    return pl.kernel(
        _spmm_body,
        mesh=mesh,
        out_type=jax.ShapeDtypeStruct((NROWS_PAD, EMB), jnp.float32),
        scratch_types=[
            pltpu.VMEM((16,), jnp.int32),
            pltpu.VMEM((16,), jnp.int32),
            pltpu.VMEM((K,), jnp.int32),
            pltpu.VMEM((K,), jnp.int32),
            pltpu.VMEM((K,), jnp.int32),
            pltpu.VMEM((K,), jnp.int32),
            pltpu.VMEM((K,), jnp.float32),
            pltpu.VMEM((K,), jnp.float32),
            pltpu.VMEM((K, EMB), jnp.float32),
            pltpu.VMEM((K, EMB), jnp.float32),
            pltpu.VMEM((RPW, EMB), jnp.float32),
        ] + [pltpu.SemaphoreType.DMA] * 10,
    )


def _spmm(dense, rows, cols, vals):
    pad = EPAD - NNZ
    rows_p = jnp.concatenate(
        [rows.astype(jnp.int32), jnp.full((pad,), NROWS_PAD, jnp.int32)])
    cols_p = jnp.concatenate([cols.astype(jnp.int32), jnp.zeros((pad,), jnp.int32)])
    vals_p = jnp.concatenate([vals, jnp.zeros((pad,), jnp.float32)])
    return _spmm_kernel(dense.shape[0])(dense, rows_p, cols_p, vals_p)


def kernel(pois_embs, hg_up_rows, hg_up_cols, hg_up_vals,
           hg_pu_rows, hg_pu_cols, hg_pu_vals):
    msg = _spmm(pois_embs, hg_up_rows, hg_up_cols, hg_up_vals)
    out = _spmm(msg, hg_pu_rows, hg_pu_cols, hg_pu_vals)
    return out[:N_POIS]


# speedup vs baseline: 3.9482x; 1.0018x over previous
import functools

import jax
import jax.numpy as jnp
from jax import lax
from jax.experimental import pallas as pl
from jax.experimental.pallas import tpu as pltpu
from jax.experimental.pallas import tpu_sc as plsc

N_POIS = 10000
N_USERS = 10000
EMB = 128
NNZ = 320000

NC = 2
NS = 16
NW = NC * NS
RPW = 320
NROWS_PAD = NW * RPW
K = 128
EPAD = NNZ + 640
SEARCH_ROUNDS = 6
LANES = EMB // 16


def _lower_bound_step(rows, probe, sem, lo, hi, target):
    step = jnp.maximum(1, (hi - lo + 15) // 16)
    q = jnp.minimum(lo + lax.iota(jnp.int32, 16) * step,
                    jnp.maximum(hi - 1, 0))
    cp = pltpu.make_async_copy(rows.at[q], probe, sem)
    cp.start()
    cp.wait()
    vals = probe[...]
    c = jnp.int32(0)
    for lane in range(16):
        c = c + jnp.where(vals[lane] < target, 1, 0).astype(jnp.int32)
    new_lo = jnp.where(c > 0, jnp.minimum(lo + (c - 1) * step + 1, hi), lo)
    new_hi = jnp.where(c < 16, jnp.minimum(lo + c * step, hi), hi)
    return new_lo, new_hi


def _spmm_body(dense, rows, cols, vals, out,
               probe_a, probe_b,
               cbuf0, cbuf1, rbuf0, rbuf1, vbuf0, vbuf1, gath0, gath1, acc,
               sem_a, sem_b,
               semc0, semc1, semr0, semr1, semv0, semv1, semg0, semg1):
    c = lax.axis_index("c")
    s = lax.axis_index("s")
    wid = s * NC + c
    r0 = wid * RPW
    r1 = r0 + RPW

    cbuf = (cbuf0, cbuf1)
    rbuf = (rbuf0, rbuf1)
    vbuf = (vbuf0, vbuf1)
    gath = (gath0, gath1)
    semc = (semc0, semc1)
    semr = (semr0, semr1)
    semv = (semv0, semv1)
    semg = (semg0, semg1)

    zero = jnp.zeros((16,), jnp.float32)

    def zero_body(r, carry):
        for cc in range(LANES):
            acc[r, pl.ds(cc * 16, 16)] = zero
        return carry

    lax.fori_loop(0, RPW, zero_body, 0)

    def bs_body(_, st):
        lo0, hi0, lo1, hi1 = st
        lo0, hi0 = _lower_bound_step(rows, probe_a, sem_a, lo0, hi0, r0)
        lo1, hi1 = _lower_bound_step(rows, probe_b, sem_b, lo1, hi1, r1)
        return lo0, hi0, lo1, hi1

    z = jnp.int32(0)
    n = jnp.int32(NNZ)
    e0, _, e1, _ = lax.fori_loop(0, SEARCH_ROUNDS, bs_body, (z, n, z, n))

    e0a = pl.multiple_of((e0 // 8) * 8, 8)
    nchunks = (e1 - e0a + (K - 1)) // K
    npairs = jnp.maximum(1, (nchunks + 1) // 2)
    iota16 = lax.iota(jnp.int32, 16)

    def cbase(ci):
        return pl.multiple_of(e0a + ci * K, 8)

    def start_cols(ci, b):
        pltpu.make_async_copy(cols.at[pl.ds(cbase(ci), K)], cbuf[b], semc[b]).start()

    def start_rv(ci, b):
        pltpu.make_async_copy(rows.at[pl.ds(cbase(ci), K)], rbuf[b], semr[b]).start()
        pltpu.make_async_copy(vals.at[pl.ds(cbase(ci), K)], vbuf[b], semv[b]).start()

    def start_gather(b):
        pltpu.make_async_copy(dense.at[cbuf[b]], gath[b], semg[b]).start()

    def wait_cols(b):
        pltpu.make_async_copy(cols.at[pl.ds(0, K)], cbuf[b], semc[b]).wait()

    def wait_rv(b):
        pltpu.make_async_copy(rows.at[pl.ds(0, K)], rbuf[b], semr[b]).wait()
        pltpu.make_async_copy(vals.at[pl.ds(0, K)], vbuf[b], semv[b]).wait()

    def wait_gather(b):
        pltpu.make_async_copy(dense.at[cbuf[b]], gath[b], semg[b]).wait()

    def accumulate(ci, b):
        base = cbase(ci)

        def group_body(g16, gcarry):
            j0 = g16 * 16
            vv = vbuf[b][pl.ds(j0, 16)]
            rr = rbuf[b][pl.ds(j0, 16)]
            eidx = base + j0 + iota16
            vvz = jnp.where((eidx >= e0) & (eidx < e1), vv, 0.0)
            rcl = jnp.clip(rr - r0, 0, RPW - 1)
            for lane in range(16):
                vb = jnp.take(vvz, jnp.full((16,), lane, jnp.int32))
                r_s = rcl[lane]
                jrow = j0 + lane
                for cc in range(LANES):
                    gv = gath[b][jrow, pl.ds(cc * 16, 16)]
                    plsc.addupdate(acc.at[r_s, pl.ds(cc * 16, 16)], gv * vb)
            return gcarry

        lax.fori_loop(0, K // 16, group_body, 0)

    start_cols(0, 0)
    wait_cols(0)
    start_gather(0)
    start_rv(0, 0)
    start_cols(1, 1)
    start_rv(1, 1)

    def pair_body(p, carry):
        ci = p * 2
        for b in (0, 1):
            wait_cols(1 - b)
            start_gather(1 - b)
            wait_gather(b)
            wait_rv(b)
            accumulate(ci + b, b)
            start_cols(ci + b + 2, b)
            start_rv(ci + b + 2, b)
        return carry

    lax.fori_loop(0, npairs, pair_body, 0)

    wait_gather(0)
    wait_cols(1)
    wait_rv(0)
    wait_rv(1)

    pltpu.sync_copy(acc, out.at[pl.ds(r0, RPW)])


@functools.cache
def _spmm_kernel(n_dense_rows):
    mesh = plsc.VectorSubcoreMesh(core_axis_name="c", subcore_axis_name="s")
    return pl.kernel(
        _spmm_body,
        mesh=mesh,
        out_type=jax.ShapeDtypeStruct((NROWS_PAD, EMB), jnp.float32),
        scratch_types=[
            pltpu.VMEM((16,), jnp.int32),
            pltpu.VMEM((16,), jnp.int32),
            pltpu.VMEM((K,), jnp.int32),
            pltpu.VMEM((K,), jnp.int32),
            pltpu.VMEM((K,), jnp.int32),
            pltpu.VMEM((K,), jnp.int32),
            pltpu.VMEM((K,), jnp.float32),
            pltpu.VMEM((K,), jnp.float32),
            pltpu.VMEM((K, EMB), jnp.float32),
            pltpu.VMEM((K, EMB), jnp.float32),
            pltpu.VMEM((RPW, EMB), jnp.float32),
        ] + [pltpu.SemaphoreType.DMA] * 10,
    )


def _spmm(dense, rows, cols, vals):
    pad = EPAD - NNZ
    rows_p = jnp.concatenate(
        [rows.astype(jnp.int32), jnp.full((pad,), NROWS_PAD, jnp.int32)])
    cols_p = jnp.concatenate([cols.astype(jnp.int32), jnp.zeros((pad,), jnp.int32)])
    vals_p = jnp.concatenate([vals, jnp.zeros((pad,), jnp.float32)])
    return _spmm_kernel(dense.shape[0])(dense, rows_p, cols_p, vals_p)


def kernel(pois_embs, hg_up_rows, hg_up_cols, hg_up_vals,
           hg_pu_rows, hg_pu_cols, hg_pu_vals):
    msg = _spmm(pois_embs, hg_up_rows, hg_up_cols, hg_up_vals)
    out = _spmm(msg, hg_pu_rows, hg_pu_cols, hg_pu_vals)
    return out[:N_POIS]


# speedup vs baseline: 10.0367x; 2.5421x over previous
import functools

import jax
import jax.numpy as jnp
from jax import lax
from jax.experimental import pallas as pl
from jax.experimental.pallas import tpu as pltpu
from jax.experimental.pallas import tpu_sc as plsc

N_POIS = 10000
EMB = 128
NNZ = 320000

NC = 2
NS = 16
NW = NC * NS
EPW = NNZ // NW
K = 128
NCHUNKS = -(-EPW // K)
NPAIRS = -(-NCHUNKS // 2)
EPAD = NNZ + 640
ROWS_OUT = 10240
SROWS = 10256
ZRPW = SROWS // 16
RPW_C = ROWS_OUT // NW
PADROW = ROWS_OUT
LANES = EMB // 16


def _scale16(gath_b, vbuf_b, base_rel, j0, e1_rel, iota16):
    vv = vbuf_b[pl.ds(j0, 16)]
    eidx = base_rel + j0 + iota16
    vvz = jnp.where(eidx < e1_rel, vv, 0.0)
    for lane in range(16):
        vb = vvz[iota16 * 0 + lane]
        jrow = j0 + lane
        for cc in range(LANES):
            sl = pl.ds(cc * 16, 16)
            gath_b[jrow, sl] = gath_b[jrow, sl] * vb
    return None


def _spmm_partial_body(dense, rows, cols, vals, out2,
                       cbuf0, cbuf1, rbuf0, rbuf1, vbuf0, vbuf1,
                       gath0, gath1, sacc,
                       semc0, semc1, semr0, semr1, semv0, semv1,
                       semg0, semg1, sems0, sems1):
    c = lax.axis_index("c")
    s = lax.axis_index("s")
    wid = c * NS + s
    e0 = wid * EPW

    cbuf = (cbuf0, cbuf1)
    rbuf = (rbuf0, rbuf1)
    vbuf = (vbuf0, vbuf1)
    gath = (gath0, gath1)
    semc = (semc0, semc1)
    semr = (semr0, semr1)
    semv = (semv0, semv1)
    semg = (semg0, semg1)
    sems = (sems0, sems1)
    iota16 = lax.iota(jnp.int32, 16)

    zero = jnp.zeros((16,), jnp.float32)

    def zb(r, carry):
        for cc in range(LANES):
            gath0[r, pl.ds(cc * 16, 16)] = zero
        return carry

    lax.fori_loop(0, K, zb, 0)
    for z in range(5):
        pltpu.sync_copy(gath0, sacc.at[pl.ds(s * ZRPW + z * K, K)])
    pltpu.sync_copy(gath0.at[pl.ds(0, 1)], sacc.at[pl.ds(s * ZRPW + 5 * K, 1)])
    plsc.subcore_barrier()

    def cbase(ci):
        return pl.multiple_of(e0 + ci * K, 8)

    def start_cols(ci, b):
        pltpu.make_async_copy(cols.at[pl.ds(cbase(ci), K)], cbuf[b], semc[b]).start()

    def start_rv(ci, b):
        pltpu.make_async_copy(rows.at[pl.ds(cbase(ci), K)], rbuf[b], semr[b]).start()
        pltpu.make_async_copy(vals.at[pl.ds(cbase(ci), K)], vbuf[b], semv[b]).start()

    def start_gather(b):
        pltpu.make_async_copy(dense.at[cbuf[b]], gath[b], semg[b]).start()

    def start_scatter(b):
        pltpu.make_async_copy(gath[b], sacc.at[rbuf[b]], sems[b]).start(add=True)

    def wait_cols(b):
        pltpu.make_async_copy(cols.at[pl.ds(0, K)], cbuf[b], semc[b]).wait()

    def wait_rv(b):
        pltpu.make_async_copy(rows.at[pl.ds(0, K)], rbuf[b], semr[b]).wait()
        pltpu.make_async_copy(vals.at[pl.ds(0, K)], vbuf[b], semv[b]).wait()

    def wait_gather(b):
        pltpu.make_async_copy(dense.at[cbuf[b]], gath[b], semg[b]).wait()

    def wait_scatter(b):
        pltpu.make_async_copy(gath[b], sacc.at[rbuf[b]], sems[b]).wait()

    def scale(ci, b):
        base_rel = ci * K

        def group_body(g16, gcarry):
            _scale16(gath[b], vbuf[b], base_rel, g16 * 16, EPW, iota16)
            return gcarry

        lax.fori_loop(0, K // 16, group_body, 0)

    def iter_steps(ci, b, first):
        if not first:
            wait_scatter(1 - b)
        wait_cols(1 - b)
        start_gather(1 - b)
        start_rv(ci + 1, 1 - b)
        wait_gather(b)
        wait_rv(b)
        scale(ci, b)
        start_scatter(b)
        start_cols(ci + 2, b)

    start_cols(0, 0)
    wait_cols(0)
    start_gather(0)
    start_rv(0, 0)
    start_cols(1, 1)

    iter_steps(0, 0, True)
    iter_steps(1, 1, False)

    def pair_body(p, carry):
        ci = p * 2
        iter_steps(ci, 0, False)
        iter_steps(ci + 1, 1, False)
        return carry

    lax.fori_loop(1, NPAIRS, pair_body, 0)

    wait_scatter(1)
    wait_gather(0)
    wait_rv(0)
    wait_cols(1)

    plsc.subcore_barrier()
    for z in range(5):
        r = s * 640 + z * K
        pltpu.sync_copy(sacc.at[pl.ds(r, K)], out2.at[c, pl.ds(r, K)])


def _combine_body(in2, out, buf0, buf1, sem0, sem1):
    c = lax.axis_index("c")
    s = lax.axis_index("s")
    wid = c * NS + s
    r0 = wid * RPW_C
    cp0 = pltpu.make_async_copy(in2.at[0, pl.ds(r0, RPW_C)], buf0, sem0)
    cp1 = pltpu.make_async_copy(in2.at[1, pl.ds(r0, RPW_C)], buf1, sem1)
    cp0.start()
    cp1.start()
    cp0.wait()
    cp1.wait()

    def add_row(r, carry):
        for cc in range(LANES):
            sl = pl.ds(cc * 16, 16)
            buf0[r, sl] = buf0[r, sl] + buf1[r, sl]
        return carry

    lax.fori_loop(0, RPW_C, add_row, 0)
    pltpu.sync_copy(buf0, out.at[pl.ds(r0, RPW_C)])


@functools.cache
def _spmm_partial_kernel(n_dense_rows):
    mesh = plsc.VectorSubcoreMesh(core_axis_name="c", subcore_axis_name="s")
    return pl.kernel(
        _spmm_partial_body,
        mesh=mesh,
        out_type=jax.ShapeDtypeStruct((NC, ROWS_OUT, EMB), jnp.float32),
        scratch_types=[
            pltpu.VMEM((K,), jnp.int32),
            pltpu.VMEM((K,), jnp.int32),
            pltpu.VMEM((K,), jnp.int32),
            pltpu.VMEM((K,), jnp.int32),
            pltpu.VMEM((K,), jnp.float32),
            pltpu.VMEM((K,), jnp.float32),
            pltpu.VMEM((K, EMB), jnp.float32),
            pltpu.VMEM((K, EMB), jnp.float32),
            pltpu.VMEM_SHARED((SROWS, EMB), jnp.float32),
        ] + [pltpu.SemaphoreType.DMA] * 10,
    )


@functools.cache
def _combine_kernel():
    mesh = plsc.VectorSubcoreMesh(core_axis_name="c", subcore_axis_name="s")
    return pl.kernel(
        _combine_body,
        mesh=mesh,
        out_type=jax.ShapeDtypeStruct((ROWS_OUT, EMB), jnp.float32),
        scratch_types=[
            pltpu.VMEM((RPW_C, EMB), jnp.float32),
            pltpu.VMEM((RPW_C, EMB), jnp.float32),
            pltpu.SemaphoreType.DMA,
            pltpu.SemaphoreType.DMA,
        ],
    )


def _spmm(dense, rows, cols, vals):
    pad = EPAD - NNZ
    rows_p = jnp.concatenate(
        [rows.astype(jnp.int32), jnp.full((pad,), PADROW, jnp.int32)])
    cols_p = jnp.concatenate([cols.astype(jnp.int32), jnp.zeros((pad,), jnp.int32)])
    vals_p = jnp.concatenate([vals, jnp.zeros((pad,), jnp.float32)])
    partials = _spmm_partial_kernel(dense.shape[0])(dense, rows_p, cols_p, vals_p)
    return _combine_kernel()(partials)


def kernel(pois_embs, hg_up_rows, hg_up_cols, hg_up_vals,
           hg_pu_rows, hg_pu_cols, hg_pu_vals):
    msg = _spmm(pois_embs, hg_up_rows, hg_up_cols, hg_up_vals)
    out = _spmm(msg, hg_pu_rows, hg_pu_cols, hg_pu_vals)
    return out[:N_POIS]


# speedup vs baseline: 10.1116x; 1.0075x over previous
import functools

import jax
import jax.numpy as jnp
from jax import lax
from jax.experimental import pallas as pl
from jax.experimental.pallas import tpu as pltpu
from jax.experimental.pallas import tpu_sc as plsc

N_POIS = 10000
EMB = 128
NNZ = 320000

NC = 2
NS = 16
NW = NC * NS
EPW = NNZ // NW
K = 112
NCHUNKS = -(-EPW // K)
NTRIS = 30
NCH_TOT = 2 + 3 * NTRIS
EPAD = NNZ + 1024
ROWS_OUT = 10240
SROWS = 10256
RPW_C = ROWS_OUT // NW
RRPW = ROWS_OUT // 16
PADROW = ROWS_OUT
LANES = EMB // 16


def _scale16(gath_b, vbuf_b, base_rel, j0, e1_rel, iota16):
    vv = vbuf_b[pl.ds(j0, 16)]
    eidx = base_rel + j0 + iota16
    vvz = jnp.where(eidx < e1_rel, vv, 0.0)
    for lane in range(16):
        vb = vvz[iota16 * 0 + lane]
        jrow = j0 + lane
        for cc in range(LANES):
            sl = pl.ds(cc * 16, 16)
            gath_b[jrow, sl] = gath_b[jrow, sl] * vb
    return None


def _spmm_partial_body(dense, rows, cols, vals, out2,
                       cbuf0, cbuf1, cbuf2, rbuf0, rbuf1, rbuf2,
                       vbuf0, vbuf1, vbuf2, gath0, gath1, gath2, sacc,
                       semc0, semc1, semc2, semr0, semr1, semr2,
                       semv0, semv1, semv2, semg0, semg1, semg2,
                       sems0, sems1, sems2):
    c = lax.axis_index("c")
    s = lax.axis_index("s")
    wid = c * NS + s
    e0 = wid * EPW

    cbuf = (cbuf0, cbuf1, cbuf2)
    rbuf = (rbuf0, rbuf1, rbuf2)
    vbuf = (vbuf0, vbuf1, vbuf2)
    gath = (gath0, gath1, gath2)
    semc = (semc0, semc1, semc2)
    semr = (semr0, semr1, semr2)
    semv = (semv0, semv1, semv2)
    semg = (semg0, semg1, semg2)
    sems = (sems0, sems1, sems2)
    iota16 = lax.iota(jnp.int32, 16)

    zero = jnp.zeros((16,), jnp.float32)

    def zb(r, carry):
        for cc in range(LANES):
            gath0[r, pl.ds(cc * 16, 16)] = zero
        return carry

    lax.fori_loop(0, K, zb, 0)
    for z in range(5):
        pltpu.sync_copy(gath0, sacc.at[pl.ds(s * RRPW + z * K, K)])
    pltpu.sync_copy(gath0.at[pl.ds(0, RRPW - 5 * K)],
                    sacc.at[pl.ds(s * RRPW + 5 * K, RRPW - 5 * K)])
    plsc.subcore_barrier()

    def cbase(ci):
        return pl.multiple_of(e0 + ci * K, 8)

    def start_cols(ci, b):
        pltpu.make_async_copy(cols.at[pl.ds(cbase(ci), K)], cbuf[b], semc[b]).start()

    def start_rv(ci, b):
        pltpu.make_async_copy(rows.at[pl.ds(cbase(ci), K)], rbuf[b], semr[b]).start()
        pltpu.make_async_copy(vals.at[pl.ds(cbase(ci), K)], vbuf[b], semv[b]).start()

    def start_gather(b):
        pltpu.make_async_copy(dense.at[cbuf[b]], gath[b], semg[b]).start()

    def start_scatter(b):
        pltpu.make_async_copy(gath[b], sacc.at[rbuf[b]], sems[b]).start(add=True)

    def wait_cols(b):
        pltpu.make_async_copy(cols.at[pl.ds(0, K)], cbuf[b], semc[b]).wait()

    def wait_rv(b):
        pltpu.make_async_copy(rows.at[pl.ds(0, K)], rbuf[b], semr[b]).wait()
        pltpu.make_async_copy(vals.at[pl.ds(0, K)], vbuf[b], semv[b]).wait()

    def wait_gather(b):
        pltpu.make_async_copy(dense.at[cbuf[b]], gath[b], semg[b]).wait()

    def wait_scatter(b):
        pltpu.make_async_copy(gath[b], sacc.at[rbuf[b]], sems[b]).wait()

    def scale(ci, b):
        base_rel = ci * K

        def group_body(g16, gcarry):
            _scale16(gath[b], vbuf[b], base_rel, g16 * 16, EPW, iota16)
            return gcarry

        lax.fori_loop(0, K // 16, group_body, 0)

    def iter_steps(ci, b, first):
        if not first:
            wait_scatter((b + 1) % 3)
        wait_cols((b + 1) % 3)
        start_gather((b + 1) % 3)
        start_rv(ci + 1, (b + 1) % 3)
        wait_gather(b)
        wait_rv(b)
        scale(ci, b)
        start_scatter(b)
        start_cols(ci + 2, (b + 2) % 3)

    start_cols(0, 0)
    wait_cols(0)
    start_gather(0)
    start_rv(0, 0)
    start_cols(1, 1)

    iter_steps(0, 0, True)
    iter_steps(1, 1, True)

    def tri_body(p, carry):
        ci = 2 + p * 3
        iter_steps(ci, 2, False)
        iter_steps(ci + 1, 0, False)
        iter_steps(ci + 2, 1, False)
        return carry

    lax.fori_loop(0, NTRIS, tri_body, 0)

    wait_scatter(0)
    wait_scatter(1)
    wait_gather(2)
    wait_rv(2)
    wait_cols(0)

    plsc.subcore_barrier()
    pltpu.sync_copy(sacc.at[pl.ds(s * RRPW, RRPW)],
                    out2.at[c, pl.ds(s * RRPW, RRPW)])


def _combine_body(in2, out, buf0, buf1, sem0, sem1):
    c = lax.axis_index("c")
    s = lax.axis_index("s")
    wid = c * NS + s
    r0 = wid * RPW_C
    cp0 = pltpu.make_async_copy(in2.at[0, pl.ds(r0, RPW_C)], buf0, sem0)
    cp1 = pltpu.make_async_copy(in2.at[1, pl.ds(r0, RPW_C)], buf1, sem1)
    cp0.start()
    cp1.start()
    cp0.wait()
    cp1.wait()

    def add_row(r, carry):
        for cc in range(LANES):
            sl = pl.ds(cc * 16, 16)
            buf0[r, sl] = buf0[r, sl] + buf1[r, sl]
        return carry

    lax.fori_loop(0, RPW_C, add_row, 0)
    pltpu.sync_copy(buf0, out.at[pl.ds(r0, RPW_C)])


@functools.cache
def _spmm_partial_kernel(n_dense_rows):
    mesh = plsc.VectorSubcoreMesh(core_axis_name="c", subcore_axis_name="s")
    return pl.kernel(
        _spmm_partial_body,
        mesh=mesh,
        out_type=jax.ShapeDtypeStruct((NC, ROWS_OUT, EMB), jnp.float32),
        scratch_types=[
            pltpu.VMEM((K,), jnp.int32),
            pltpu.VMEM((K,), jnp.int32),
            pltpu.VMEM((K,), jnp.int32),
            pltpu.VMEM((K,), jnp.int32),
            pltpu.VMEM((K,), jnp.int32),
            pltpu.VMEM((K,), jnp.int32),
            pltpu.VMEM((K,), jnp.float32),
            pltpu.VMEM((K,), jnp.float32),
            pltpu.VMEM((K,), jnp.float32),
            pltpu.VMEM((K, EMB), jnp.float32),
            pltpu.VMEM((K, EMB), jnp.float32),
            pltpu.VMEM((K, EMB), jnp.float32),
            pltpu.VMEM_SHARED((SROWS, EMB), jnp.float32),
        ] + [pltpu.SemaphoreType.DMA] * 15,
    )


@functools.cache
def _combine_kernel():
    mesh = plsc.VectorSubcoreMesh(core_axis_name="c", subcore_axis_name="s")
    return pl.kernel(
        _combine_body,
        mesh=mesh,
        out_type=jax.ShapeDtypeStruct((ROWS_OUT, EMB), jnp.float32),
        scratch_types=[
            pltpu.VMEM((RPW_C, EMB), jnp.float32),
            pltpu.VMEM((RPW_C, EMB), jnp.float32),
            pltpu.SemaphoreType.DMA,
            pltpu.SemaphoreType.DMA,
        ],
    )


def _spmm(dense, rows, cols, vals):
    pad = EPAD - NNZ
    rows_p = jnp.concatenate(
        [rows.astype(jnp.int32), jnp.full((pad,), PADROW, jnp.int32)])
    cols_p = jnp.concatenate([cols.astype(jnp.int32), jnp.zeros((pad,), jnp.int32)])
    vals_p = jnp.concatenate([vals, jnp.zeros((pad,), jnp.float32)])
    partials = _spmm_partial_kernel(dense.shape[0])(dense, rows_p, cols_p, vals_p)
    return _combine_kernel()(partials)


def kernel(pois_embs, hg_up_rows, hg_up_cols, hg_up_vals,
           hg_pu_rows, hg_pu_cols, hg_pu_vals):
    msg = _spmm(pois_embs, hg_up_rows, hg_up_cols, hg_up_vals)
    out = _spmm(msg, hg_pu_rows, hg_pu_cols, hg_pu_vals)
    return out[:N_POIS]
